# Initial kernel scaffold; baseline (speedup 1.0000x reference)
import jax
import jax.numpy as jnp
from jax.experimental import pallas as pl


def kernel(logits, labels):
    raise NotImplementedError("write your pallas kernel here")


# speedup vs baseline: 1.3060x; 1.3060x over previous
import functools

import jax
import jax.numpy as jnp
from jax.experimental import pallas as pl

_BINS = 15
_N = 16384
_C = 1000
_R = 1024

_EDGES = [
    0.0, 0.06666667014360428, 0.13333334028720856, 0.20000001788139343,
    0.2666666805744171, 0.3333333432674408, 0.40000003576278687,
    0.46666669845581055, 0.5333333611488342, 0.6000000238418579,
    0.6666666865348816, 0.7333333492279053, 0.8000000715255737,
    0.8666667342185974, 0.9333333969116211, 1.0,
]


def _ece_block(logits_ref, labels_ref, cnt_ref, csum_ref, asum_ref, ece_ref):
    step = pl.program_id(0)
    nsteps = pl.num_programs(0)

    @pl.when(step == 0)
    def _init():
        cnt_ref[...] = jnp.zeros_like(cnt_ref)
        csum_ref[...] = jnp.zeros_like(csum_ref)
        asum_ref[...] = jnp.zeros_like(asum_ref)
        ece_ref[...] = jnp.zeros_like(ece_ref)

    x = logits_ref[...]
    labels = labels_ref[0, 0, :]

    m = jnp.max(x, axis=-1, keepdims=True)
    s = jnp.sum(jnp.exp(x - m), axis=-1)
    p = 1.0 / s
    col = jax.lax.broadcasted_iota(jnp.int32, x.shape, 1)
    pred = jnp.min(jnp.where(x == m, col, _C), axis=-1)
    correct = (pred == labels).astype(jnp.float32)

    lane = jax.lax.broadcasted_iota(jnp.int32, (1, 128), 1)
    lo_row = jnp.full((1, 128), 2.0, dtype=jnp.float32)
    hi_row = jnp.full((1, 128), 3.0, dtype=jnp.float32)
    for i in range(_BINS):
        lo_row = jnp.where(lane == i, _EDGES[i], lo_row)
        hi_row = jnp.where(lane == i, _EDGES[i + 1], hi_row)
    pd = p[:, None]
    onehot = ((pd > lo_row) & (pd <= hi_row)).astype(jnp.float32)
    cnt_ref[...] += jnp.sum(onehot, axis=0, keepdims=True)
    csum_ref[...] += jnp.sum(onehot * p[:, None], axis=0, keepdims=True)
    asum_ref[...] += jnp.sum(onehot * correct[:, None], axis=0, keepdims=True)

    @pl.when(step == nsteps - 1)
    def _finish():
        cnt = cnt_ref[...]
        csum = csum_ref[...]
        asum = asum_ref[...]
        safe = jnp.where(cnt > 0, cnt, 1.0)
        e = jnp.where(cnt > 0, csum / safe - asum / safe, 0.0)
        ece_ref[...] = jnp.sum(jnp.abs(e) * (cnt / _N)).reshape(1, 1)


@jax.jit
def _ece(logits, labels):
    grid = _N // _R
    labels3 = labels.astype(jnp.int32).reshape(grid, 1, _R)
    out = pl.pallas_call(
        _ece_block,
        grid=(grid,),
        in_specs=[
            pl.BlockSpec((_R, _C), lambda i: (i, 0)),
            pl.BlockSpec((1, 1, _R), lambda i: (i, 0, 0)),
        ],
        out_specs=[
            pl.BlockSpec((1, 128), lambda i: (0, 0)),
            pl.BlockSpec((1, 128), lambda i: (0, 0)),
            pl.BlockSpec((1, 128), lambda i: (0, 0)),
            pl.BlockSpec((1, 1), lambda i: (0, 0)),
        ],
        out_shape=[
            jax.ShapeDtypeStruct((1, 128), jnp.float32),
            jax.ShapeDtypeStruct((1, 128), jnp.float32),
            jax.ShapeDtypeStruct((1, 128), jnp.float32),
            jax.ShapeDtypeStruct((1, 1), jnp.float32),
        ],
    )(logits, labels3)
    return out[3][0, 0]


def kernel(logits, labels):
    return _ece(logits, labels)


# speedup vs baseline: 1.3524x; 1.0356x over previous
import functools

import jax
import jax.numpy as jnp
from jax.experimental import pallas as pl

_BINS = 15
_N = 16384
_C = 1000
_R = 2048

_EDGES = [
    0.0, 0.06666667014360428, 0.13333334028720856, 0.20000001788139343,
    0.2666666805744171, 0.3333333432674408, 0.40000003576278687,
    0.46666669845581055, 0.5333333611488342, 0.6000000238418579,
    0.6666666865348816, 0.7333333492279053, 0.8000000715255737,
    0.8666667342185974, 0.9333333969116211, 1.0,
]


def _ece_block(logits_ref, labels_ref, cnt_ref, csum_ref, asum_ref, ece_ref):
    step = pl.program_id(0)
    nsteps = pl.num_programs(0)

    @pl.when(step == 0)
    def _init():
        cnt_ref[...] = jnp.zeros_like(cnt_ref)
        csum_ref[...] = jnp.zeros_like(csum_ref)
        asum_ref[...] = jnp.zeros_like(asum_ref)
        ece_ref[...] = jnp.zeros_like(ece_ref)

    x = logits_ref[...]
    labels = labels_ref[0, 0, :]

    m = jnp.max(x, axis=-1, keepdims=True)
    s = jnp.sum(jnp.exp(x - m), axis=-1)
    p = 1.0 / s
    col = jax.lax.broadcasted_iota(jnp.int32, x.shape, 1)
    pred = jnp.min(jnp.where(x == m, col, _C), axis=-1)
    correct = (pred == labels).astype(jnp.float32)

    lane = jax.lax.broadcasted_iota(jnp.int32, (1, 128), 1)
    lo_row = jnp.full((1, 128), 2.0, dtype=jnp.float32)
    hi_row = jnp.full((1, 128), 3.0, dtype=jnp.float32)
    for i in range(_BINS):
        lo_row = jnp.where(lane == i, _EDGES[i], lo_row)
        hi_row = jnp.where(lane == i, _EDGES[i + 1], hi_row)
    pd = p[:, None]
    onehot = ((pd > lo_row) & (pd <= hi_row)).astype(jnp.float32)
    cnt_ref[...] += jnp.sum(onehot, axis=0, keepdims=True)
    csum_ref[...] += jnp.sum(onehot * p[:, None], axis=0, keepdims=True)
    asum_ref[...] += jnp.sum(onehot * correct[:, None], axis=0, keepdims=True)

    @pl.when(step == nsteps - 1)
    def _finish():
        cnt = cnt_ref[...]
        csum = csum_ref[...]
        asum = asum_ref[...]
        safe = jnp.where(cnt > 0, cnt, 1.0)
        e = jnp.where(cnt > 0, csum / safe - asum / safe, 0.0)
        ece_ref[...] = jnp.sum(jnp.abs(e) * (cnt / _N)).reshape(1, 1)


@jax.jit
def _ece(logits, labels):
    grid = _N // _R
    labels3 = labels.astype(jnp.int32).reshape(grid, 1, _R)
    out = pl.pallas_call(
        _ece_block,
        grid=(grid,),
        in_specs=[
            pl.BlockSpec((_R, _C), lambda i: (i, 0)),
            pl.BlockSpec((1, 1, _R), lambda i: (i, 0, 0)),
        ],
        out_specs=[
            pl.BlockSpec((1, 128), lambda i: (0, 0)),
            pl.BlockSpec((1, 128), lambda i: (0, 0)),
            pl.BlockSpec((1, 128), lambda i: (0, 0)),
            pl.BlockSpec((1, 1), lambda i: (0, 0)),
        ],
        out_shape=[
            jax.ShapeDtypeStruct((1, 128), jnp.float32),
            jax.ShapeDtypeStruct((1, 128), jnp.float32),
            jax.ShapeDtypeStruct((1, 128), jnp.float32),
            jax.ShapeDtypeStruct((1, 1), jnp.float32),
        ],
    )(logits, labels3)
    return out[3][0, 0]


def kernel(logits, labels):
    return _ece(logits, labels)
